# baseline (speedup 1.0000x reference)
import functools

import jax
import jax.numpy as jnp
from jax import lax
from jax.experimental import pallas as pl
from jax.experimental.pallas import tpu as pltpu
from jax.experimental.pallas import tpu_sc as plsc

VOCAB = 1000
VPAD = 1024
NPC = VPAD // 128
NB = 1024
NS_SEQ = 200
NTOK = NB * NS_SEQ

NC = 2
NSUB = 16
NW = NC * NSUB
PER_W = NTOK // NW
CKT = 16
NCH = PER_W // CKT
NBUF = 4
LANES = 16


def _lse_tc(table):
    def body(t_ref, o_ref):
        x = t_ref[...]
        m = jnp.max(x, axis=1, keepdims=True)
        s = jnp.sum(jnp.exp(x - m), axis=1, keepdims=True)
        o_ref[...] = jnp.log(s) + m

    return pl.pallas_call(
        body,
        out_shape=jax.ShapeDtypeStruct((VOCAB, 1), jnp.float32),
    )(table)


@functools.partial(
    pl.kernel,
    out_type=[
        jax.ShapeDtypeStruct((NTOK * NPC, 128), jnp.float32),
        jax.ShapeDtypeStruct((NW, LANES), jnp.float32),
    ],
    mesh=plsc.VectorSubcoreMesh(core_axis_name="c", subcore_axis_name="s"),
    compiler_params=pltpu.CompilerParams(
        needs_layout_passes=False, use_tc_tiling_on_sc=True
    ),
    scratch_types=[
        pltpu.VMEM((NCH, 128), jnp.int32),
        pltpu.VMEM((PER_W,), jnp.int32),
        pltpu.VMEM((NBUF * 128, 128), jnp.float32),
        pltpu.VMEM((VPAD,), jnp.float32),
        pltpu.VMEM((LANES,), jnp.float32),
        pltpu.SemaphoreType.DMA,
        pltpu.SemaphoreType.DMA,
        pltpu.SemaphoreType.DMA,
        pltpu.SemaphoreType.DMA,
        pltpu.SemaphoreType.DMA,
        pltpu.SemaphoreType.DMA,
        pltpu.SemaphoreType.DMA,
        pltpu.SemaphoreType.DMA,
    ],
)
def _sc_gather(tsub_hbm, sidx_hbm, tgt_hbm, lse_hbm,
               out_hbm, loss_hbm,
               sidx_v, tgt_v, rows_v, lse_v, part_v,
               g0, g1, g2, g3, s0, s1, s2, s3):
    wid = lax.axis_index("s") * NC + lax.axis_index("c")
    gsem = (g0, g1, g2, g3)
    ssem = (s0, s1, s2, s3)

    pltpu.sync_copy(lse_hbm, lse_v)
    pltpu.sync_copy(sidx_hbm.at[wid], sidx_v)
    pltpu.sync_copy(tgt_hbm.at[wid], tgt_v)

    def rbuf(b):
        return rows_v.at[pl.ds(b * 128, 128)]

    def gather_start(c, b):
        pltpu.async_copy(tsub_hbm.at[sidx_v.at[c]], rbuf(b), gsem[b])

    def gather_wait(c, b):
        pltpu.make_async_copy(
            tsub_hbm.at[sidx_v.at[c]], rbuf(b), gsem[b]).wait()

    def out_slice(c):
        return out_hbm.at[pl.ds((wid * PER_W + c * CKT) * NPC, CKT * NPC)]

    def scatter_start(c, b):
        pltpu.async_copy(rbuf(b), out_slice(c), ssem[b])

    def scatter_wait(c, b):
        pltpu.make_async_copy(rbuf(b), out_slice(c), ssem[b]).wait()

    iota = lax.iota(jnp.int32, LANES)

    def compute(c, b, acc):
        row16 = jnp.broadcast_to(c, (LANES,))
        i8 = plsc.load_gather(sidx_v, [row16, iota * NPC])
        inp16 = lax.shift_right_logical(i8, 3)
        lse16 = plsc.load_gather(lse_v, [inp16])
        tgt16 = tgt_v[pl.ds(c * CKT, LANES)]
        rid16 = iota * NPC + lax.shift_right_logical(tgt16, 7)
        lane16 = lax.bitwise_and(tgt16, 127)
        tv = plsc.load_gather(rbuf(b), [rid16, lane16])
        return acc + (lse16 - tv)

    def slot(c, b, acc, head=False, tail=False):
        gather_wait(c, b)
        scatter_start(c, b)
        acc = compute(c, b, acc)
        if not tail:
            if not head:
                scatter_wait(c - 2, (b + 2) % NBUF)
            gather_start(c + 2, (b + 2) % NBUF)
        return acc

    acc = jnp.zeros((LANES,), jnp.float32)
    gather_start(0, 0)
    gather_start(1, 1)
    acc = slot(0, 0, acc, head=True)
    acc = slot(1, 1, acc, head=True)

    def body(i, acc):
        c = NBUF * i
        for u in range(NBUF):
            acc = slot(c + u + 2, (u + 2) % NBUF, acc)
        return acc

    acc = lax.fori_loop(0, (NCH - 4) // NBUF, body, acc)
    acc = slot(NCH - 2, (NCH - 2) % NBUF, acc, tail=True)
    acc = slot(NCH - 1, (NCH - 1) % NBUF, acc, tail=True)
    for c in range(NCH - 4, NCH):
        scatter_wait(c, c % NBUF)

    part_v[...] = acc
    pltpu.sync_copy(part_v, loss_hbm.at[wid])


def _rearrange_tc(raw):
    GRP = 5
    SEQB = 8 * GRP
    CPB = NS_SEQ // SEQB

    def body(in_ref, o_ref):
        for g in range(GRP):
            for j in range(NPC):
                w = min(128, VOCAB - j * 128)
                o_ref[0, pl.ds(g * 8, 8), pl.ds(j * 128, w)] = (
                    in_ref[g, :, j, :w])

    in4 = raw.reshape(NTOK // 8, 8, NPC, 128)
    grid = (NTOK // SEQB,)
    return pl.pallas_call(
        body,
        grid=grid,
        in_specs=[pl.BlockSpec((GRP, 8, NPC, 128),
                               lambda i: (i, 0, 0, 0))],
        out_specs=pl.BlockSpec((1, SEQB, VOCAB),
                               lambda i: (i // CPB, i % CPB, 0)),
        out_shape=jax.ShapeDtypeStruct((NB, NS_SEQ, VOCAB), jnp.float32),
    )(in4)


def kernel(input, target, table):
    lse = _lse_tc(table).reshape(VOCAB)
    lsep = jnp.pad(lse, (0, VPAD - VOCAB))
    tsub = jnp.pad(table, ((0, 0), (0, VPAD - VOCAB))).reshape(
        VOCAB * NPC, 128)
    sidx = (input.reshape(-1, 1) * NPC
            + jnp.arange(NPC, dtype=jnp.int32)).reshape(NW, NCH, 128)
    tgt = target.reshape(NW, PER_W)
    raw, parts = _sc_gather(tsub, sidx, tgt, lsep)
    logits = _rearrange_tc(raw)
    loss = jnp.sum(parts) / jnp.float32(NTOK)
    return (logits, loss)


# speedup vs baseline: 2.1227x; 2.1227x over previous
import functools

import jax
import jax.numpy as jnp
from jax import lax
from jax.experimental import pallas as pl
from jax.experimental.pallas import tpu as pltpu
from jax.experimental.pallas import tpu_sc as plsc

VOCAB = 1000
NB = 1024
NS_SEQ = 200
NTOK = NB * NS_SEQ

NC = 2
NSUB = 16
NW = NC * NSUB
PER_W = NTOK // NW
NB_W = PER_W // NS_SEQ
CK = 40
CPB = NS_SEQ // CK
NCH = PER_W // CK
LANES = 16


def _lse_tc(table):
    def body(t_ref, o_ref):
        x = t_ref[...]
        m = jnp.max(x, axis=1, keepdims=True)
        s = jnp.sum(jnp.exp(x - m), axis=1, keepdims=True)
        o_ref[...] = jnp.log(s) + m

    return pl.pallas_call(
        body,
        out_shape=jax.ShapeDtypeStruct((VOCAB, 1), jnp.float32),
    )(table)


@functools.partial(
    pl.kernel,
    out_type=[
        jax.ShapeDtypeStruct((NB, NS_SEQ, VOCAB), jnp.float32),
        jax.ShapeDtypeStruct((NW, LANES), jnp.float32),
    ],
    mesh=plsc.VectorSubcoreMesh(core_axis_name="c", subcore_axis_name="s"),
    compiler_params=pltpu.CompilerParams(
        needs_layout_passes=False, use_tc_tiling_on_sc=False
    ),
    scratch_types=[
        pltpu.VMEM((PER_W + LANES,), jnp.int32),
        pltpu.VMEM((PER_W + LANES,), jnp.int32),
        pltpu.VMEM((CK, VOCAB), jnp.float32),
        pltpu.VMEM((CK, VOCAB), jnp.float32),
        pltpu.VMEM((VOCAB,), jnp.float32),
        pltpu.VMEM((LANES,), jnp.float32),
        pltpu.SemaphoreType.DMA,
        pltpu.SemaphoreType.DMA,
        pltpu.SemaphoreType.DMA,
        pltpu.SemaphoreType.DMA,
    ],
)
def _sc_gather(table_hbm, idx_hbm, tgt_hbm, lse_hbm,
               out_hbm, loss_hbm,
               idx_all, tgt_all, rows0, rows1, lse_v, part_v,
               gsem0, gsem1, ssem0, ssem1):
    wid = lax.axis_index("s") * NC + lax.axis_index("c")
    rows = (rows0, rows1)
    gsem = (gsem0, gsem1)
    ssem = (ssem0, ssem1)

    pltpu.sync_copy(lse_hbm, lse_v)
    pltpu.sync_copy(idx_hbm.at[wid], idx_all.at[pl.ds(0, PER_W)])
    pltpu.sync_copy(tgt_hbm.at[wid], tgt_all.at[pl.ds(0, PER_W)])

    def out_slice(c):
        batch = wid * NB_W + c // CPB
        s0 = (c % CPB) * CK
        return out_hbm.at[batch, pl.ds(s0, CK)]

    def gather_start(c, b):
        pltpu.async_copy(
            table_hbm.at[idx_all.at[pl.ds(c * CK, CK)]], rows[b], gsem[b])

    def gather_wait(c, b):
        pltpu.make_async_copy(
            table_hbm.at[idx_all.at[pl.ds(c * CK, CK)]], rows[b], gsem[b]
        ).wait()

    def scatter_start(c, b):
        pltpu.async_copy(rows[b], out_slice(c), ssem[b])

    def scatter_wait(c, b):
        pltpu.make_async_copy(rows[b], out_slice(c), ssem[b]).wait()

    iota = lax.iota(jnp.int32, LANES)
    tail_mask = iota < (CK % LANES)

    def compute(c, b, acc):
        base = c * CK
        for j in range(-(-CK // LANES)):
            off = base + j * LANES
            inp16 = idx_all[pl.ds(off, LANES)]
            tgt16 = tgt_all[pl.ds(off, LANES)]
            rid = iota + (j * LANES)
            if (j + 1) * LANES <= CK:
                lse16 = plsc.load_gather(lse_v, [inp16])
                tv = plsc.load_gather(rows[b], [rid, tgt16])
                acc = acc + (lse16 - tv)
            else:
                lse16 = plsc.load_gather(lse_v, [inp16], mask=tail_mask)
                tv = plsc.load_gather(rows[b], [rid, tgt16], mask=tail_mask)
                acc = acc + jnp.where(tail_mask, lse16 - tv, 0.0)
        return acc

    def slot(c, b, acc, first=False, last=False):
        if not first:
            scatter_wait(c - 1, 1 - b)
        if not last:
            gather_start(c + 1, 1 - b)
        gather_wait(c, b)
        scatter_start(c, b)
        return compute(c, b, acc)

    acc = jnp.zeros((LANES,), jnp.float32)
    gather_start(0, 0)
    acc = slot(0, 0, acc, first=True)

    def body(i, acc):
        acc = slot(2 * i + 1, 1, acc)
        acc = slot(2 * i + 2, 0, acc)
        return acc

    acc = lax.fori_loop(0, (NCH - 2) // 2, body, acc)
    acc = slot(NCH - 1, 1, acc, last=True)
    scatter_wait(NCH - 1, 1)

    part_v[...] = acc
    pltpu.sync_copy(part_v, loss_hbm.at[wid])


def kernel(input, target, table):
    lse = _lse_tc(table).reshape(VOCAB)
    idx = input.reshape(NW, PER_W)
    tgt = target.reshape(NW, PER_W)
    logits, parts = _sc_gather(table, idx, tgt, lse)
    loss = jnp.sum(parts) / jnp.float32(NTOK)
    return (logits, loss)


# speedup vs baseline: 2.1495x; 1.0126x over previous
import functools

import jax
import jax.numpy as jnp
from jax import lax
from jax.experimental import pallas as pl
from jax.experimental.pallas import tpu as pltpu
from jax.experimental.pallas import tpu_sc as plsc

VOCAB = 1000
VPAD = 1024
NB = 1024
NS_SEQ = 200
NTOK = NB * NS_SEQ

NC = 2
NSUB = 16
NW = NC * NSUB
PER_W = NTOK // NW
NB_W = PER_W // NS_SEQ
CK = 40
CPB = NS_SEQ // CK
NCH = PER_W // CK
LANES = 16


def _lse_tc(table):
    def body(t_ref, o_ref):
        x = t_ref[...]
        m = jnp.max(x, axis=1, keepdims=True)
        s = jnp.sum(jnp.exp(x - m), axis=1, keepdims=True)
        o_ref[...] = jnp.log(s) + m

    return pl.pallas_call(
        body,
        out_shape=jax.ShapeDtypeStruct((VOCAB, 1), jnp.float32),
    )(table)


@functools.partial(
    pl.kernel,
    out_type=[
        jax.ShapeDtypeStruct((NB, NS_SEQ, VPAD), jnp.float32),
        jax.ShapeDtypeStruct((NW, LANES), jnp.float32),
    ],
    mesh=plsc.VectorSubcoreMesh(core_axis_name="c", subcore_axis_name="s"),
    compiler_params=pltpu.CompilerParams(
        needs_layout_passes=False, use_tc_tiling_on_sc=False
    ),
    scratch_types=[
        pltpu.VMEM((PER_W + LANES,), jnp.int32),
        pltpu.VMEM((PER_W + LANES,), jnp.int32),
        pltpu.VMEM((CK, VPAD), jnp.float32),
        pltpu.VMEM((CK, VPAD), jnp.float32),
        pltpu.VMEM((VOCAB,), jnp.float32),
        pltpu.VMEM((LANES,), jnp.float32),
        pltpu.SemaphoreType.DMA,
        pltpu.SemaphoreType.DMA,
        pltpu.SemaphoreType.DMA,
        pltpu.SemaphoreType.DMA,
    ],
)
def _sc_gather(table_hbm, idx_hbm, tgt_hbm, lse_hbm,
               out_hbm, loss_hbm,
               idx_all, tgt_all, rows0, rows1, lse_v, part_v,
               gsem0, gsem1, ssem0, ssem1):
    wid = lax.axis_index("s") * NC + lax.axis_index("c")
    rows = (rows0, rows1)
    gsem = (gsem0, gsem1)
    ssem = (ssem0, ssem1)

    pltpu.sync_copy(lse_hbm, lse_v)
    pltpu.sync_copy(idx_hbm.at[wid], idx_all.at[pl.ds(0, PER_W)])
    pltpu.sync_copy(tgt_hbm.at[wid], tgt_all.at[pl.ds(0, PER_W)])

    def out_slice(c):
        batch = wid * NB_W + c // CPB
        s0 = (c % CPB) * CK
        return out_hbm.at[batch, pl.ds(s0, CK)]

    def gather_start(c, b):
        pltpu.async_copy(
            table_hbm.at[idx_all.at[pl.ds(c * CK, CK)]], rows[b], gsem[b])

    def gather_wait(c, b):
        pltpu.make_async_copy(
            table_hbm.at[idx_all.at[pl.ds(c * CK, CK)]], rows[b], gsem[b]
        ).wait()

    def scatter_start(c, b):
        pltpu.async_copy(rows[b], out_slice(c), ssem[b])

    def scatter_wait(c, b):
        pltpu.make_async_copy(rows[b], out_slice(c), ssem[b]).wait()

    iota = lax.iota(jnp.int32, LANES)
    tail_mask = iota < (CK % LANES)

    def compute(c, b, acc):
        base = c * CK
        for j in range(-(-CK // LANES)):
            off = base + j * LANES
            inp16 = idx_all[pl.ds(off, LANES)]
            tgt16 = tgt_all[pl.ds(off, LANES)]
            rid = iota + (j * LANES)
            if (j + 1) * LANES <= CK:
                lse16 = plsc.load_gather(lse_v, [inp16])
                tv = plsc.load_gather(rows[b], [rid, tgt16])
                acc = acc + (lse16 - tv)
            else:
                lse16 = plsc.load_gather(lse_v, [inp16], mask=tail_mask)
                tv = plsc.load_gather(rows[b], [rid, tgt16], mask=tail_mask)
                acc = acc + jnp.where(tail_mask, lse16 - tv, 0.0)
        return acc

    def slot(c, b, acc, first=False, last=False):
        if not first:
            scatter_wait(c - 1, 1 - b)
        if not last:
            gather_start(c + 1, 1 - b)
        gather_wait(c, b)
        scatter_start(c, b)
        return compute(c, b, acc)

    acc = jnp.zeros((LANES,), jnp.float32)
    gather_start(0, 0)
    acc = slot(0, 0, acc, first=True)

    def body(i, acc):
        acc = slot(2 * i + 1, 1, acc)
        acc = slot(2 * i + 2, 0, acc)
        return acc

    acc = lax.fori_loop(0, (NCH - 2) // 2, body, acc)
    acc = slot(NCH - 1, 1, acc, last=True)
    scatter_wait(NCH - 1, 1)

    part_v[...] = acc
    pltpu.sync_copy(part_v, loss_hbm.at[wid])


def kernel(input, target, table):
    lse = _lse_tc(table).reshape(VOCAB)
    tpad = jnp.pad(table, ((0, 0), (0, VPAD - VOCAB)))
    idx = input.reshape(NW, PER_W)
    tgt = target.reshape(NW, PER_W)
    padded, parts = _sc_gather(tpad, idx, tgt, lse)
    logits = padded[:, :, :VOCAB]
    loss = jnp.sum(parts) / jnp.float32(NTOK)
    return (logits, loss)
